# Initial kernel scaffold; baseline (speedup 1.0000x reference)
import jax
import jax.numpy as jnp
from jax.experimental import pallas as pl


def kernel(x, edge_index, edge_attr, Wx, bx, W, b, gamma, beta):
    raise NotImplementedError("write your pallas kernel here")


# speedup vs baseline: 1.1855x; 1.1855x over previous
import dataclasses
import functools

import jax
import jax.numpy as jnp
from jax import lax
from jax.experimental import pallas as pl
from jax.experimental.pallas import tpu as pltpu
from jax.experimental.pallas import tpu_sc as plsc

_N = 10000
_E = 160000
_DIN = 256
_D = 512
_L = 3
_EPS = 1e-5

_CW = 128
_NCHUNK = _D // _CW
_NC = 2
_NS = 16
_NW = _NC * _NS
_RPT = 640
_RPT_LAST = _N - 15 * _RPT
_ACCR = 656
_PADROW = 648

_SEG = 4000
_NSEG = _E // _SEG
_LISTCAP = 4096
_GB = 128

_EPT_H = _E // _NW
_HV = _EPT_H // 16

_NODE_BLK = 1000

_mesh = plsc.VectorSubcoreMesh(core_axis_name="c", subcore_axis_name="s")

_sc_params = pltpu.CompilerParams()
if "needs_layout_passes" in pltpu.CompilerParams.__dataclass_fields__:
    _sc_params = dataclasses.replace(_sc_params, needs_layout_passes=False)


def _hist_body(dst_hbm, out_hbm, dbuf, acc):
    c = lax.axis_index("c")
    s = lax.axis_index("s")
    w = c * _NS + s
    iota = lax.broadcasted_iota(jnp.int32, (16,), 0)
    ones = jnp.ones((16,), jnp.float32)

    @pl.loop(0, _N // 16)
    def _(i):
        acc[pl.ds(i * 16, 16)] = jnp.zeros((16,), jnp.float32)

    pltpu.sync_copy(dst_hbm.at[pl.ds(w * _EPT_H, _EPT_H)],
                    dbuf.at[pl.ds(0, _EPT_H)])

    @pl.loop(0, _HV)
    def _(v):
        idx = dbuf[pl.ds(v * 16, 16)]
        for j in range(16):
            plsc.addupdate_scatter(acc, [idx], ones, mask=iota == j)

    tidx = dbuf[pl.ds(_HV * 16, 16)]
    for j in range(8):
        plsc.addupdate_scatter(acc, [tidx], ones, mask=iota == j)

    pltpu.sync_copy(acc, out_hbm.at[w])


@functools.partial(
    pl.kernel,
    out_type=jax.ShapeDtypeStruct((_NW, _N), jnp.float32),
    mesh=_mesh,
    scratch_types=[
        pltpu.VMEM((_EPT_H + 16,), jnp.int32),
        pltpu.VMEM((_N,), jnp.float32),
    ],
    compiler_params=_sc_params,
)
def _sc_hist(dst_hbm, out_hbm, dbuf, acc):
    _hist_body(dst_hbm, out_hbm, dbuf, acc)


def _agg_body(g_hbm, src_hbm, dst_hbm, out_hbm,
              sbuf, dbuf, srclist, dstlist, rows, acc, sem):
    c = lax.axis_index("c")
    s = lax.axis_index("s")
    lo = s * _RPT
    iota = lax.broadcasted_iota(jnp.int32, (16,), 0)

    for j in range(_NCHUNK // _NC):
        chunk = c * (_NCHUNK // _NC) + j
        goff = chunk * _N

        @pl.loop(0, _ACCR)
        def _(i):
            for k in range(8):
                acc[i, pl.ds(k * 16, 16)] = jnp.zeros((16,), jnp.float32)

        @pl.loop(0, _NSEG)
        def _(seg):
            e0 = seg * _SEG
            pltpu.sync_copy(src_hbm.at[pl.ds(e0, _SEG)],
                            sbuf.at[pl.ds(0, _SEG)])
            pltpu.sync_copy(dst_hbm.at[pl.ds(e0, _SEG)],
                            dbuf.at[pl.ds(0, _SEG)])

            @pl.loop(0, _LISTCAP // 16)
            def _(v):
                srclist[pl.ds(v * 16, 16)] = jnp.full((16,), goff, jnp.int32)
                dstlist[pl.ds(v * 16, 16)] = jnp.full((16,), _PADROW,
                                                      jnp.int32)

            def scan_step(v, cnt):
                dv = dbuf[pl.ds(v * 16, 16)]
                sv = sbuf[pl.ds(v * 16, 16)]
                m = (dv >= lo) & (dv < lo + _RPT)
                pcs = plsc.cumsum(m.astype(jnp.int32))
                pos = pcs + (cnt - 1)
                plsc.store_scatter(srclist, [pos], sv + goff, mask=m)
                plsc.store_scatter(dstlist, [pos], dv - lo, mask=m)
                return cnt + jnp.max(pcs)

            cnt = lax.fori_loop(0, _SEG // 16, scan_step, jnp.int32(0))
            nb = (cnt + (_GB - 1)) // _GB

            def drain(b, _):
                pltpu.async_copy(
                    g_hbm.at[srclist.at[pl.ds(b * _GB, _GB)]], rows,
                    sem).wait()

                def row_add(rg, _2):
                    dl_v = dstlist[pl.ds(b * _GB + rg * 16, 16)]
                    for r2 in range(16):
                        rsplat = jnp.full((16,), dl_v[r2], jnp.int32)
                        r = rg * 16 + r2
                        for k in range(8):
                            plsc.addupdate_scatter(
                                acc, [rsplat, iota + (k * 16)],
                                rows[r, pl.ds(k * 16, 16)])
                    return _2

                return lax.fori_loop(0, _GB // 16, row_add, _)

            lax.fori_loop(0, nb, drain, jnp.int32(0))

        @pl.when(s < _NS - 1)
        def _():
            pltpu.sync_copy(acc.at[pl.ds(0, _RPT)],
                            out_hbm.at[pl.ds(goff + lo, _RPT)])

        @pl.when(s == _NS - 1)
        def _():
            pltpu.sync_copy(acc.at[pl.ds(0, _RPT_LAST)],
                            out_hbm.at[pl.ds(goff + lo, _RPT_LAST)])


@functools.partial(
    pl.kernel,
    out_type=jax.ShapeDtypeStruct((_NCHUNK * _N, _CW), jnp.float32),
    mesh=_mesh,
    scratch_types=[
        pltpu.VMEM((_SEG,), jnp.int32),
        pltpu.VMEM((_SEG,), jnp.int32),
        pltpu.VMEM((_LISTCAP,), jnp.int32),
        pltpu.VMEM((_LISTCAP,), jnp.int32),
        pltpu.VMEM((_GB, _CW), jnp.float32),
        pltpu.VMEM((_ACCR, _CW), jnp.float32),
        pltpu.SemaphoreType.DMA,
    ],
    compiler_params=_sc_params,
)
def _sc_agg(g_hbm, src_hbm, dst_hbm, out_hbm,
            sbuf, dbuf, srclist, dstlist, rows, acc, sem):
    _agg_body(g_hbm, src_hbm, dst_hbm, out_hbm,
              sbuf, dbuf, srclist, dstlist, rows, acc, sem)


def _prep_body(x_ref, wx_ref, bx_ref, w0_ref, hist_ref,
               h0_ref, g0_ref, dinv_ref):
    deg = jnp.sum(hist_ref[0], axis=1) + 1.0
    dinv = lax.rsqrt(deg)[:, None]
    h0 = jnp.dot(x_ref[...], wx_ref[...],
                 preferred_element_type=jnp.float32) + bx_ref[...]
    h0_ref[...] = h0
    hw = jnp.dot(h0, w0_ref[...], preferred_element_type=jnp.float32)
    g = hw * dinv
    dinv_ref[...] = jnp.broadcast_to(dinv, (_NODE_BLK, _CW))
    for cc in range(_NCHUNK):
        g0_ref[cc] = g[:, cc * _CW:(cc + 1) * _CW]


def _tc_prep(x, Wx, bx2, W0, hist):
    grid = (_N // _NODE_BLK,)
    return pl.pallas_call(
        _prep_body,
        grid=grid,
        in_specs=[
            pl.BlockSpec((_NODE_BLK, _DIN), lambda i: (i, 0)),
            pl.BlockSpec((_DIN, _D), lambda i: (0, 0)),
            pl.BlockSpec((1, _D), lambda i: (0, 0)),
            pl.BlockSpec((_D, _D), lambda i: (0, 0)),
            pl.BlockSpec((1, _NODE_BLK, _NW), lambda i: (i, 0, 0)),
        ],
        out_specs=[
            pl.BlockSpec((_NODE_BLK, _D), lambda i: (i, 0)),
            pl.BlockSpec((_NCHUNK, _NODE_BLK, _CW), lambda i: (0, i, 0)),
            pl.BlockSpec((_NODE_BLK, _CW), lambda i: (i, 0)),
        ],
        out_shape=[
            jax.ShapeDtypeStruct((_N, _D), jnp.float32),
            jax.ShapeDtypeStruct((_NCHUNK, _N, _CW), jnp.float32),
            jax.ShapeDtypeStruct((_N, _CW), jnp.float32),
        ],
    )(x, Wx, bx2, W0, hist)


def _post_body(s_ref, g_ref, dinv_ref, hp_ref, b_ref, t_ref, st_ref):
    i = pl.program_id(0)

    @pl.when(i == 0)
    def _():
        st_ref[...] = jnp.zeros_like(st_ref)

    dv = dinv_ref[:, 0:1]
    for cc in range(_NCHUNK):
        sl = slice(cc * _CW, (cc + 1) * _CW)
        tcc = (s_ref[cc] + g_ref[cc]) * dv + b_ref[0:1, sl] + hp_ref[:, sl]
        t_ref[:, sl] = tcc
        st_ref[0:1, sl] += jnp.sum(tcc, axis=0, keepdims=True)
        st_ref[1:2, sl] += jnp.sum(tcc * tcc, axis=0, keepdims=True)


def _tc_post(s4, g4, dinv, hprev, b2):
    grid = (_N // _NODE_BLK,)
    return pl.pallas_call(
        _post_body,
        grid=grid,
        in_specs=[
            pl.BlockSpec((_NCHUNK, _NODE_BLK, _CW), lambda i: (0, i, 0)),
            pl.BlockSpec((_NCHUNK, _NODE_BLK, _CW), lambda i: (0, i, 0)),
            pl.BlockSpec((_NODE_BLK, _CW), lambda i: (i, 0)),
            pl.BlockSpec((_NODE_BLK, _D), lambda i: (i, 0)),
            pl.BlockSpec((1, _D), lambda i: (0, 0)),
        ],
        out_specs=[
            pl.BlockSpec((_NODE_BLK, _D), lambda i: (i, 0)),
            pl.BlockSpec((2, _D), lambda i: (0, 0)),
        ],
        out_shape=[
            jax.ShapeDtypeStruct((_N, _D), jnp.float32),
            jax.ShapeDtypeStruct((2, _D), jnp.float32),
        ],
    )(s4, g4, dinv, hprev, b2)


def _next_body(t_ref, st_ref, gm_ref, bt_ref, wn_ref, dinv_ref,
               hn_ref, gn_ref):
    mu = st_ref[0:1, :] * (1.0 / _N)
    var = st_ref[1:2, :] * (1.0 / _N) - mu * mu
    rs = lax.rsqrt(var + _EPS)
    a = (t_ref[...] - mu) * (rs * gm_ref[...]) + bt_ref[...]
    a = jnp.maximum(a, 0.0)
    hn_ref[...] = a
    hw = jnp.dot(a, wn_ref[...], preferred_element_type=jnp.float32)
    g = hw * dinv_ref[:, 0:1]
    for cc in range(_NCHUNK):
        gn_ref[cc] = g[:, cc * _CW:(cc + 1) * _CW]


def _tc_next(t, st, gm2, bt2, Wn, dinv):
    grid = (_N // _NODE_BLK,)
    return pl.pallas_call(
        _next_body,
        grid=grid,
        in_specs=[
            pl.BlockSpec((_NODE_BLK, _D), lambda i: (i, 0)),
            pl.BlockSpec((2, _D), lambda i: (0, 0)),
            pl.BlockSpec((1, _D), lambda i: (0, 0)),
            pl.BlockSpec((1, _D), lambda i: (0, 0)),
            pl.BlockSpec((_D, _D), lambda i: (0, 0)),
            pl.BlockSpec((_NODE_BLK, _CW), lambda i: (i, 0)),
        ],
        out_specs=[
            pl.BlockSpec((_NODE_BLK, _D), lambda i: (i, 0)),
            pl.BlockSpec((_NCHUNK, _NODE_BLK, _CW), lambda i: (0, i, 0)),
        ],
        out_shape=[
            jax.ShapeDtypeStruct((_N, _D), jnp.float32),
            jax.ShapeDtypeStruct((_NCHUNK, _N, _CW), jnp.float32),
        ],
    )(t, st, gm2, bt2, Wn, dinv)


def _final_body(t_ref, st_ref, gm_ref, bt_ref, o_ref):
    mu = st_ref[0:1, :] * (1.0 / _N)
    var = st_ref[1:2, :] * (1.0 / _N) - mu * mu
    rs = lax.rsqrt(var + _EPS)
    o_ref[...] = (t_ref[...] - mu) * (rs * gm_ref[...]) + bt_ref[...]


def _tc_final(t, st, gm2, bt2):
    grid = (_N // _NODE_BLK,)
    return pl.pallas_call(
        _final_body,
        grid=grid,
        in_specs=[
            pl.BlockSpec((_NODE_BLK, _D), lambda i: (i, 0)),
            pl.BlockSpec((2, _D), lambda i: (0, 0)),
            pl.BlockSpec((1, _D), lambda i: (0, 0)),
            pl.BlockSpec((1, _D), lambda i: (0, 0)),
        ],
        out_specs=pl.BlockSpec((_NODE_BLK, _D), lambda i: (i, 0)),
        out_shape=jax.ShapeDtypeStruct((_N, _D), jnp.float32),
    )(t, st, gm2, bt2)


def kernel(x, edge_index, edge_attr, Wx, bx, W, b, gamma, beta):
    src = edge_index[0]
    dst = edge_index[1]

    hist = _sc_hist(dst)
    hist_t = hist.reshape(_NW, _N // _NODE_BLK, _NODE_BLK).transpose(1, 2, 0)

    h, g4, dinv = _tc_prep(x, Wx, bx.reshape(1, _D), W[0], hist_t)
    out = None
    for l in range(_L):
        s_flat = _sc_agg(g4.reshape(_NCHUNK * _N, _CW), src, dst)
        s4 = s_flat.reshape(_NCHUNK, _N, _CW)
        t, st = _tc_post(s4, g4, dinv, h, b[l].reshape(1, _D))
        if l < _L - 1:
            h, g4 = _tc_next(t, st, gamma[l].reshape(1, _D),
                             beta[l].reshape(1, _D), W[l + 1], dinv)
        else:
            out = _tc_final(t, st, gamma[l].reshape(1, _D),
                            beta[l].reshape(1, _D))
    return out
